# baseline (speedup 1.0000x reference)
import jax
import jax.numpy as jnp
from jax import lax
from jax.experimental import pallas as pl


def _restore_body(x0_ref, acc_ref, out_ref):
    x0 = x0_ref[...]
    out_ref[...] = jnp.where(x0 != 0, x0, acc_ref[...])


def kernel(x, adj_indices, adj_values, mask, iter):
    n, d = x.shape
    row = adj_indices[0]
    col = adj_indices[1]
    x0 = jnp.where(mask != 0, x, 0.0)

    restore = pl.pallas_call(
        _restore_body,
        out_shape=jax.ShapeDtypeStruct((n, d), x.dtype),
    )

    def body(i, out):
        g = out[col] * adj_values[:, None]
        acc = jax.ops.segment_sum(g, row, num_segments=n)
        return restore(x0, acc)

    return lax.fori_loop(0, iter, body, x)


# speedup vs baseline: 2.4829x; 2.4829x over previous
import jax
import jax.numpy as jnp
from jax import lax
from jax.experimental import pallas as pl
from jax.experimental.pallas import tpu as pltpu
from jax.experimental.pallas import tpu_sc as plsc

N_PAD = 10240
RPW = 320
NW = 32
D = 128
CH = 128
BLK = 1024
NCH = BLK // CH


def _spmm_body(xp, x0p, cs, vs, rs, starts, out,
               cs_v, vs_v, rs_v, st_v, g_buf, acc, x0_buf, sem):
    c = lax.axis_index("c")
    s = lax.axis_index("s")
    w = s * 2 + c
    lo = w * RPW
    hi = lo + RPW
    lane = lax.iota(jnp.int32, 16)

    pltpu.sync_copy(starts, st_v)
    wv = jnp.full((16,), w, jnp.int32)
    s_raw = jnp.max(plsc.load_gather(st_v, [wv]))
    e_raw = jnp.max(plsc.load_gather(st_v, [wv + 1]))
    s0 = pl.multiple_of(lax.bitwise_and(s_raw, jnp.int32(-8)), 8)

    def zbody(i, carry):
        acc[pl.ds(i * 16, 16)] = jnp.zeros((16,), jnp.float32)
        return carry
    lax.fori_loop(0, (RPW * D) // 16, zbody, 0)

    nblk = lax.div(e_raw - s0 + (BLK - 1), jnp.int32(BLK))

    def blk_body(b, carry):
        off = pl.multiple_of(s0 + b * BLK, 8)
        pltpu.sync_copy(cs.at[pl.ds(off, BLK)], cs_v)
        pltpu.sync_copy(vs.at[pl.ds(off, BLK)], vs_v)
        pltpu.sync_copy(rs.at[pl.ds(off, BLK)], rs_v)
        rem = e_raw - off
        kmax = jnp.clip(lax.div(rem + (CH - 1), jnp.int32(CH)), 0, NCH)

        def ch_body(k, carry2):
            pltpu.async_copy(xp.at[cs_v.at[pl.ds(k * CH, CH)]], g_buf, sem).wait()

            def e_body(i, carry3):
                iv = jnp.full((16,), k * CH + i, jnp.int32)
                vbc = plsc.load_gather(vs_v, [iv])
                rbc = plsc.load_gather(rs_v, [iv])
                m = (rbc >= lo) & (rbc < hi)
                veff = jnp.where(m, vbc, jnp.zeros((16,), jnp.float32))
                lr = jnp.clip(rbc - lo, 0, RPW - 1)
                ob = lr * D + lane
                ivv = jnp.full((16,), i, jnp.int32)
                for f in range(D // 16):
                    gv = plsc.load_gather(g_buf, [ivv, lane + f * 16])
                    plsc.addupdate_scatter(acc, [ob + f * 16], gv * veff)
                return carry3
            lax.fori_loop(0, CH, e_body, 0)
            return carry2
        lax.fori_loop(0, kmax, ch_body, 0)
        return carry
    lax.fori_loop(0, nblk, blk_body, 0)

    pltpu.sync_copy(x0p.at[pl.ds(pl.multiple_of(lo * D, 8), RPW * D)], x0_buf)

    def r_body(i, carry):
        x0v = x0_buf[pl.ds(i * 16, 16)]
        a = acc[pl.ds(i * 16, 16)]
        acc[pl.ds(i * 16, 16)] = jnp.where(x0v != 0.0, x0v, a)
        return carry
    lax.fori_loop(0, (RPW * D) // 16, r_body, 0)

    pltpu.sync_copy(acc, out.at[pl.ds(pl.multiple_of(lo * D, 8), RPW * D)])


def kernel(x, adj_indices, adj_values, mask, iter):
    n, d = x.shape
    row = adj_indices[0].astype(jnp.int32)
    col = adj_indices[1].astype(jnp.int32)
    vals = adj_values.astype(jnp.float32)
    e = row.shape[0]

    order = jnp.argsort(row)
    rs = row[order]
    cs = col[order]
    vs = vals[order]
    ep = e + 2 * BLK
    rs_p = jnp.full((ep,), 1 << 20, jnp.int32).at[:e].set(rs)
    cs_p = jnp.zeros((ep,), jnp.int32).at[:e].set(cs)
    vs_p = jnp.zeros((ep,), jnp.float32).at[:e].set(vs)
    bounds = jnp.arange(33, dtype=jnp.int32) * RPW
    starts = jnp.searchsorted(rs, bounds).astype(jnp.int32)
    starts = jnp.concatenate([starts, jnp.full((15,), e, jnp.int32)])

    x0 = jnp.where(mask != 0, x.astype(jnp.float32), 0.0)
    x0_flat = jnp.zeros((N_PAD * d,), jnp.float32).at[: n * d].set(x0.reshape(-1))
    x_pad = jnp.zeros((N_PAD, d), jnp.float32).at[:n].set(x.astype(jnp.float32))

    step = pl.kernel(
        _spmm_body,
        out_type=jax.ShapeDtypeStruct((N_PAD * d,), jnp.float32),
        mesh=plsc.VectorSubcoreMesh(core_axis_name="c", subcore_axis_name="s"),
        compiler_params=pltpu.CompilerParams(needs_layout_passes=False),
        scratch_types=[
            pltpu.VMEM((BLK,), jnp.int32),
            pltpu.VMEM((BLK,), jnp.float32),
            pltpu.VMEM((BLK,), jnp.int32),
            pltpu.VMEM((48,), jnp.int32),
            pltpu.VMEM((CH, D), jnp.float32),
            pltpu.VMEM((RPW * D,), jnp.float32),
            pltpu.VMEM((RPW * D,), jnp.float32),
            pltpu.SemaphoreType.DMA,
        ],
    )

    def body(i, xp):
        return step(xp, x0_flat, cs_p, vs_p, rs_p, starts).reshape(N_PAD, d)

    xf = lax.fori_loop(0, iter, body, x_pad)
    return xf[:n].astype(x.dtype)


# speedup vs baseline: 3.2750x; 1.3190x over previous
import jax
import jax.numpy as jnp
from jax import lax
from jax.experimental import pallas as pl
from jax.experimental.pallas import tpu as pltpu
from jax.experimental.pallas import tpu_sc as plsc

NC = 2
NS = 16
NW = NC * NS
N_PAD = 10240
RPS = N_PAD // NS
RPW = N_PAD // NW
D = 128
CH = 64
NCH = 16
BLK = CH * NCH


def _accum_body(xp, cs2, vs2, rs2, out,
                acc, csb, vsb, rsb, g0, g1, g2,
                sg0, sg1, sg2, ss0, ss1, ss2):
    c = lax.axis_index("c")
    s = lax.axis_index("s")
    w = s * NC + c
    lane = lax.iota(jnp.int32, 16)
    nblk = cs2.shape[0] // (NW * NCH)
    rowbase0 = w * (nblk * NCH)
    gbufs = (g0, g1, g2)
    sgs = (sg0, sg1, sg2)
    sss = (ss0, ss1, ss2)
    zeros16 = jnp.zeros((16,), jnp.float32)

    def zb_body(i, carry):
        r = jnp.full((16,), i // 8, jnp.int32)
        col = jnp.full((16,), (i % 8) * 16, jnp.int32) + lane
        plsc.store_scatter(g0, [r, col], zeros16)
        return carry
    lax.fori_loop(0, (CH * D) // 16, zb_body, 0)
    for j in range(RPS // CH):
        pltpu.sync_copy(g0, acc.at[pl.ds(s * RPS + j * CH, CH)])
    plsc.subcore_barrier()

    def blk_body(b, carry):
        rowbase = rowbase0 + b * NCH
        pltpu.sync_copy(cs2.at[pl.ds(rowbase, NCH)], csb)
        pltpu.sync_copy(vs2.at[pl.ds(rowbase, NCH)], vsb)
        pltpu.sync_copy(rs2.at[pl.ds(rowbase, NCH)], rsb)

        gather_pending = {}
        scatter_pending = {}
        gather_pending[0] = pltpu.async_copy(
            xp.at[csb.at[0]], gbufs[0], sgs[0])

        for k in range(NCH):
            a = k % 3
            if k + 1 < NCH:
                na = (k + 1) % 3
                if na in scatter_pending:
                    scatter_pending.pop(na).wait()
                gather_pending[na] = pltpu.async_copy(
                    xp.at[csb.at[k + 1]], gbufs[na], sgs[na])
            gather_pending.pop(a).wait()

            g = gbufs[a]
            kvec = jnp.full((16,), k, jnp.int32)

            def e_body(i, carry2):
                base = jnp.full((16,), i * 4, jnp.int32)
                for u in range(4):
                    iv = base + u
                    vbc = plsc.load_gather(vsb, [kvec, iv])
                    for f in range(D // 16):
                        cf = lane + (f * 16)
                        gv = plsc.load_gather(g, [iv, cf])
                        plsc.store_scatter(g, [iv, cf], gv * vbc)
                return carry2
            lax.fori_loop(0, CH // 4, e_body, 0)

            scatter_pending[a] = pltpu.async_copy(
                g, acc.at[rsb.at[k]], sss[a], add=True)
        for a in sorted(scatter_pending):
            scatter_pending.pop(a).wait()
        return carry
    lax.fori_loop(0, nblk, blk_body, 0)

    plsc.subcore_barrier()
    out_pending = {}
    for j in range(RPS // CH):
        a = j % 2
        if a in out_pending:
            out_pending.pop(a).wait()
        gb = gbufs[a]
        pltpu.sync_copy(acc.at[pl.ds(s * RPS + j * CH, CH)], gb)
        out_pending[a] = pltpu.async_copy(
            gb, out.at[c].at[pl.ds(s * RPS + j * CH, CH)], sgs[a])
    for a in sorted(out_pending):
        out_pending.pop(a).wait()


def _combine_body(pf, x0f, xnf, bufa, bufb, bufx, s0, s1, s2):
    c = lax.axis_index("c")
    s = lax.axis_index("s")
    w = s * NC + c
    off = pl.multiple_of(w * (RPW * D), 8)
    cpa = pltpu.async_copy(pf.at[0].at[pl.ds(off, RPW * D)], bufa, s0)
    cpb = pltpu.async_copy(pf.at[1].at[pl.ds(off, RPW * D)], bufb, s1)
    cpx = pltpu.async_copy(x0f.at[pl.ds(off, RPW * D)], bufx, s2)
    cpa.wait()
    cpb.wait()
    cpx.wait()

    def r_body(i, carry):
        for u in range(2):
            ds = pl.ds((i * 2 + u) * 16, 16)
            av = bufa[ds]
            bv = bufb[ds]
            xv = bufx[ds]
            bufa[ds] = jnp.where(xv != 0.0, xv, av + bv)
        return carry
    lax.fori_loop(0, (RPW * D) // 32, r_body, 0)
    pltpu.sync_copy(bufa, xnf.at[pl.ds(off, RPW * D)])


def kernel(x, adj_indices, adj_values, mask, iter):
    n, d = x.shape
    row = adj_indices[0].astype(jnp.int32)
    col = adj_indices[1].astype(jnp.int32)
    vals = adj_values.astype(jnp.float32)
    e = row.shape[0]

    ep = ((e + NW * BLK - 1) // (NW * BLK)) * (NW * BLK)
    cs2 = jnp.zeros((ep,), jnp.int32).at[:e].set(col).reshape(ep // CH, CH)
    rs2 = jnp.zeros((ep,), jnp.int32).at[:e].set(row).reshape(ep // CH, CH)
    vs2 = jnp.zeros((ep,), jnp.float32).at[:e].set(vals).reshape(ep // CH, CH)

    x0 = jnp.where(mask != 0, x.astype(jnp.float32), 0.0)
    x0f = jnp.zeros((N_PAD * d,), jnp.float32).at[: n * d].set(x0.reshape(-1))
    x_pad = jnp.zeros((N_PAD, d), jnp.float32).at[:n].set(x.astype(jnp.float32))

    accum = pl.kernel(
        _accum_body,
        out_type=jax.ShapeDtypeStruct((NC, N_PAD, D), jnp.float32),
        mesh=plsc.VectorSubcoreMesh(core_axis_name="c", subcore_axis_name="s"),
        compiler_params=pltpu.CompilerParams(needs_layout_passes=False),
        scratch_types=[
            pltpu.VMEM_SHARED((N_PAD, D), jnp.float32),
            pltpu.VMEM((NCH, CH), jnp.int32),
            pltpu.VMEM((NCH, CH), jnp.float32),
            pltpu.VMEM((NCH, CH), jnp.int32),
            pltpu.VMEM((CH, D), jnp.float32),
            pltpu.VMEM((CH, D), jnp.float32),
            pltpu.VMEM((CH, D), jnp.float32),
            pltpu.SemaphoreType.DMA,
            pltpu.SemaphoreType.DMA,
            pltpu.SemaphoreType.DMA,
            pltpu.SemaphoreType.DMA,
            pltpu.SemaphoreType.DMA,
            pltpu.SemaphoreType.DMA,
        ],
    )

    combine = pl.kernel(
        _combine_body,
        out_type=jax.ShapeDtypeStruct((N_PAD * D,), jnp.float32),
        mesh=plsc.VectorSubcoreMesh(core_axis_name="c", subcore_axis_name="s"),
        compiler_params=pltpu.CompilerParams(needs_layout_passes=False),
        scratch_types=[
            pltpu.VMEM((RPW * D,), jnp.float32),
            pltpu.VMEM((RPW * D,), jnp.float32),
            pltpu.VMEM((RPW * D,), jnp.float32),
            pltpu.SemaphoreType.DMA,
            pltpu.SemaphoreType.DMA,
            pltpu.SemaphoreType.DMA,
        ],
    )

    def body(i, xp):
        partials = accum(xp, cs2, vs2, rs2)
        xnf = combine(partials.reshape(NC, N_PAD * D), x0f)
        return xnf.reshape(N_PAD, D)

    xf = lax.fori_loop(0, iter, body, x_pad)
    return xf[:n].astype(x.dtype)
